# Initial kernel scaffold; baseline (speedup 1.0000x reference)
import jax
import jax.numpy as jnp
from jax.experimental import pallas as pl


def kernel(node_embeddings, edge_index, W_in, b_in, W_out, b_out, W_g, b_g, W_e1, b_e1, W_e2, b_e2, W_e3, b_e3, alpha, w_mat, d_vec):
    raise NotImplementedError("write your pallas kernel here")


# speedup vs baseline: 6.9463x; 6.9463x over previous
import functools

import jax
import jax.numpy as jnp
from jax import lax
from jax.experimental import pallas as pl
from jax.experimental.pallas import tpu as pltpu
from jax.experimental.pallas import tpu_sc as plsc

N = 10000
E = 160000
D = 128
D_IN = 64
K = 128
NCHUNKS = E // K
NC = 2
NS = 16
NW = NC * NS
N_PAD = 10112
NT = N_PAD // NS
L = 16
DT = 0.5

f32 = jnp.float32
i32 = jnp.int32

BN = 400
NB = N // BN
EB = 4000
NEB = E // EB


def _mesh():
    return plsc.VectorSubcoreMesh(core_axis_name="c", subcore_axis_name="s")


def _wid():
    cid = lax.axis_index("c")
    sid = lax.axis_index("s")
    return cid, sid, sid * NC + cid


def _zero_vmem2d(buf, nrows, ncols):
    zv = jnp.zeros((L,), f32)

    def row(i, c):
        for g in range(ncols // L):
            buf[i, pl.ds(g * L, L)] = zv
        return c

    lax.fori_loop(0, nrows, row, 0)


def _zero_shared_slice(accum, zbuf, sid, zrows):
    base = sid * NT
    off = 0
    while off < NT:
        n = min(zrows, NT - off)
        pltpu.sync_copy(zbuf.at[pl.ds(0, n)], accum.at[pl.ds(base + off, n)])
        off += n


def _sc_gather_add_body(xa, xb, src, dst, out, idx_s, idx_d, ra, rb, sem_a, sem_b):
    cid, sid, wid = _wid()
    nt = (NCHUNKS - wid + NW - 1) // NW

    def chunk(t, carry):
        base = (wid + t * NW) * K
        pltpu.sync_copy(src.at[pl.ds(base, K)], idx_s)
        pltpu.sync_copy(dst.at[pl.ds(base, K)], idx_d)
        ca = pltpu.async_copy(xa.at[idx_s], ra, sem_a)
        cb = pltpu.async_copy(xb.at[idx_d], rb, sem_b)
        ca.wait()
        cb.wait()

        def row(i, c2):
            for g in range(D // L):
                s = pl.ds(g * L, L)
                ra[i, s] = ra[i, s] + rb[i, s]
            return c2

        lax.fori_loop(0, K, row, 0)
        pltpu.sync_copy(ra, out.at[pl.ds(base, K)])
        return carry

    lax.fori_loop(0, nt, chunk, 0)


_sc_gather_add = functools.partial(
    pl.kernel,
    out_type=jax.ShapeDtypeStruct((E, D), f32),
    mesh=_mesh(),
    scratch_types=[
        pltpu.VMEM((K,), i32),
        pltpu.VMEM((K,), i32),
        pltpu.VMEM((K, D), f32),
        pltpu.VMEM((K, D), f32),
        pltpu.SemaphoreType.DMA,
        pltpu.SemaphoreType.DMA,
    ],
)(_sc_gather_add_body)


def _sc_deg_body(weighted, *refs):
    if weighted:
        w_rows, dst, out, idx_d, rowbuf, zbuf, accum = refs
    else:
        dst, out, idx_d, rowbuf, zbuf, accum = refs
    cid, sid, wid = _wid()
    nt = (NCHUNKS - wid + NW - 1) // NW

    _zero_vmem2d(zbuf, K, L)
    if not weighted:
        ones = jnp.ones((L,), f32)

        def orow(i, c):
            rowbuf[i, :] = ones
            return c

        lax.fori_loop(0, K, orow, 0)
    _zero_shared_slice(accum, zbuf, sid, K)
    plsc.subcore_barrier()

    def chunk(t, carry):
        base = (wid + t * NW) * K
        pltpu.sync_copy(dst.at[pl.ds(base, K)], idx_d)
        if weighted:
            pltpu.sync_copy(w_rows.at[pl.ds(base, K)], rowbuf)
        pltpu.sync_copy(rowbuf, accum.at[idx_d], add=True)
        return carry

    lax.fori_loop(0, nt, chunk, 0)
    plsc.subcore_barrier()
    pltpu.sync_copy(accum.at[pl.ds(sid * NT, NT)],
                    out.at[pl.ds(cid * N_PAD + sid * NT, NT)])


def _make_sc_deg(weighted):
    scratch = [
        pltpu.VMEM((K,), i32),
        pltpu.VMEM((K, L), f32),
        pltpu.VMEM((K, L), f32),
        pltpu.VMEM_SHARED((N_PAD, L), f32),
    ]
    return functools.partial(
        pl.kernel,
        out_type=jax.ShapeDtypeStruct((NC * N_PAD, L), f32),
        mesh=_mesh(),
        scratch_types=scratch,
    )(functools.partial(_sc_deg_body, weighted))


_sc_deg_w = _make_sc_deg(True)
_sc_count = _make_sc_deg(False)


def _sc_msg_body(use_ev, dw, *refs):
    if use_ev:
        table, ev_rows, src, dst, out, idx_s, idx_d, evr, rows, zbuf, sem, accum = refs
    else:
        table, src, dst, out, idx_s, idx_d, rows, zbuf, sem, accum = refs
    cid, sid, wid = _wid()
    nt = (NCHUNKS - wid + NW - 1) // NW

    _zero_vmem2d(zbuf, K, dw)
    _zero_shared_slice(accum, zbuf, sid, K)
    plsc.subcore_barrier()

    def chunk(t, carry):
        base = (wid + t * NW) * K
        pltpu.sync_copy(src.at[pl.ds(base, K)], idx_s)
        pltpu.sync_copy(dst.at[pl.ds(base, K)], idx_d)
        if use_ev:
            pltpu.sync_copy(ev_rows.at[pl.ds(base, K)], evr)
        pltpu.async_copy(table.at[idx_s], rows, sem).wait()
        if use_ev:
            def row(i, c2):
                sv = evr[i, :]
                for g in range(dw // L):
                    s = pl.ds(g * L, L)
                    rows[i, s] = rows[i, s] * sv
                return c2

            lax.fori_loop(0, K, row, 0)
        pltpu.sync_copy(rows, accum.at[idx_d], add=True)
        return carry

    lax.fori_loop(0, nt, chunk, 0)
    plsc.subcore_barrier()
    pltpu.sync_copy(accum.at[pl.ds(sid * NT, NT)],
                    out.at[pl.ds(cid * N_PAD + sid * NT, NT)])


def _make_sc_msg(use_ev, dw):
    scratch = [pltpu.VMEM((K,), i32), pltpu.VMEM((K,), i32)]
    if use_ev:
        scratch.append(pltpu.VMEM((K, L), f32))
    scratch += [
        pltpu.VMEM((K, dw), f32),
        pltpu.VMEM((K, dw), f32),
        pltpu.SemaphoreType.DMA,
        pltpu.VMEM_SHARED((N_PAD, dw), f32),
    ]
    return functools.partial(
        pl.kernel,
        out_type=jax.ShapeDtypeStruct((NC * N_PAD, dw), f32),
        mesh=_mesh(),
        scratch_types=scratch,
    )(functools.partial(_sc_msg_body, use_ev, dw))


_sc_msg_ev = _make_sc_msg(True, D)
_sc_msg_plain = _make_sc_msg(False, D)


def _full(shape):
    return pl.BlockSpec(shape, lambda i: tuple(0 for _ in shape))


def _rows(b, width):
    return pl.BlockSpec((b, width), lambda i: (i, 0))


def _ln(z):
    mu = jnp.mean(z, axis=-1, keepdims=True)
    zc = z - mu
    var = jnp.mean(zc * zc, axis=-1, keepdims=True)
    return zc * lax.rsqrt(var + 1e-5)


def _silu(x):
    return x * jax.nn.sigmoid(x)


def _dot(a, b):
    return jnp.dot(a, b, preferred_element_type=f32)


def _tc_w2_body(wm_ref, dv_ref, out_ref):
    wm = wm_ref[...]
    dcl = jnp.clip(dv_ref[...], 0.0, 1.0)
    out_ref[...] = lax.dot_general(wm * dcl, wm, (((1,), (1,)), ((), ())),
                                   preferred_element_type=f32)


_tc_w2 = pl.pallas_call(
    _tc_w2_body,
    grid=(1,),
    in_specs=[_full((D, D)), _full((1, D))],
    out_specs=_full((D, D)),
    out_shape=jax.ShapeDtypeStruct((D, D), f32),
)


def _tc_a_body(z_ref, w1a_ref, w1b_ref, wg_ref, be1_ref, xa_ref, xb_ref, xw_ref):
    z = z_ref[...]
    xa_ref[...] = _dot(z, w1a_ref[...]) + be1_ref[...]
    xb_ref[...] = _dot(z, w1b_ref[...])
    xw_ref[...] = _dot(_ln(z), wg_ref[...])


_tc_a = pl.pallas_call(
    _tc_a_body,
    grid=(NB,),
    in_specs=[_rows(BN, D), _full((D, D)), _full((D, D)), _full((D, D)),
              _full((1, D))],
    out_specs=[_rows(BN, D)] * 3,
    out_shape=[jax.ShapeDtypeStruct((N, D), f32)] * 3,
)


def _tc_edge_body(hs_ref, we2_ref, be2_ref, w3_ref, be3_ref, ev_ref):
    h = _silu(hs_ref[...])
    u = _dot(h, we2_ref[...]) + be2_ref[...]
    u = _silu(u)
    t = jnp.sum(u * w3_ref[...], axis=-1, keepdims=True) + be3_ref[0]
    ev_ref[...] = jnp.broadcast_to(jax.nn.sigmoid(t), (t.shape[0], L))


_tc_edge = pl.pallas_call(
    _tc_edge_body,
    grid=(NEB,),
    in_specs=[_rows(EB, D), _full((D, D_IN)), _full((1, D_IN)),
              _full((1, D_IN)),
              pl.BlockSpec(memory_space=pltpu.SMEM)],
    out_specs=_rows(EB, L),
    out_shape=jax.ShapeDtypeStruct((E, L), f32),
)


def _deg_to_dinv(degp):
    deg = 1.0 + degp[0, :, 0:1] + degp[1, :, 0:1]
    return lax.rsqrt(deg)


def _tc_din_body(emb_ref, win_ref, degp_ref, xws_ref, dinv_ref):
    dinv = _deg_to_dinv(degp_ref[...])
    xws_ref[...] = dinv * _dot(emb_ref[...], win_ref[...])
    dinv_ref[...] = dinv


_tc_din = pl.pallas_call(
    _tc_din_body,
    grid=(NB,),
    in_specs=[_rows(BN, D_IN), _full((D_IN, D)),
              pl.BlockSpec((NC, BN, L), lambda i: (0, i, 0))],
    out_specs=[_rows(BN, D), _rows(BN, 1)],
    out_shape=[jax.ShapeDtypeStruct((N, D), f32),
               jax.ShapeDtypeStruct((N, 1), f32)],
)


def _tc_in_post_body(aggp_ref, xws_ref, dinv_ref, bin_ref, x0_ref):
    aggp = aggp_ref[...]
    t = dinv_ref[...] * (aggp[0] + aggp[1] + xws_ref[...]) + bin_ref[...]
    nrm = jnp.sqrt(jnp.sum(t * t, axis=-1, keepdims=True))
    x0_ref[...] = t / jnp.maximum(nrm, 1e-12)


_tc_in_post = pl.pallas_call(
    _tc_in_post_body,
    grid=(NB,),
    in_specs=[pl.BlockSpec((NC, BN, D), lambda i: (0, i, 0)),
              _rows(BN, D), _rows(BN, 1), _full((1, D))],
    out_specs=_rows(BN, D),
    out_shape=jax.ShapeDtypeStruct((N, D), f32),
)


def _tc_c_body(degp_ref, xw_ref, dinv_ref, xws_ref):
    dinv = _deg_to_dinv(degp_ref[...])
    dinv_ref[...] = dinv
    xws_ref[...] = dinv * xw_ref[...]


_tc_c = pl.pallas_call(
    _tc_c_body,
    grid=(NB,),
    in_specs=[pl.BlockSpec((NC, BN, L), lambda i: (0, i, 0)), _rows(BN, D)],
    out_specs=[_rows(BN, 1), _rows(BN, D)],
    out_shape=[jax.ShapeDtypeStruct((N, 1), f32),
               jax.ShapeDtypeStruct((N, D), f32)],
)


def _tc_d_body(stage, with_a, *refs):
    if stage == 0:
        (z_ref, cur_ref, aggp_ref, dinv_ref, xws_ref, al_ref, w2_ref, bg_ref,
         *rest) = refs
        acc_ref = None
    else:
        (z_ref, cur_ref, acc_ref, aggp_ref, dinv_ref, xws_ref, al_ref, w2_ref,
         bg_ref, *rest) = refs
    if with_a:
        w1a_ref, w1b_ref, wg_ref, be1_ref = rest[:4]
        outs = rest[4:]
        y_ref, accn_ref, xa_ref, xb_ref, xw_ref = outs
    else:
        y_ref, accn_ref = rest

    z = z_ref[...]
    aggp = aggp_ref[...]
    gn = dinv_ref[...] * (aggp[0] + aggp[1] + xws_ref[...]) + bg_ref[...] \
        - _ln(z)
    a2 = jax.nn.sigmoid(al_ref[...]) * 0.5
    k = a2 * gn - 2.0 * z + _dot(z, w2_ref[...])
    if stage == 0:
        accn = k
    elif stage in (1, 2):
        accn = acc_ref[...] + 2.0 * k
    else:
        accn = acc_ref[...] + k
    accn_ref[...] = accn
    cur = cur_ref[...]
    if stage < 3:
        y = cur + (DT / 2.0 if stage < 2 else DT) * k
    else:
        y = cur + (DT / 6.0) * accn
    y_ref[...] = y
    if with_a:
        xa_ref[...] = _dot(y, w1a_ref[...]) + be1_ref[...]
        xb_ref[...] = _dot(y, w1b_ref[...])
        xw_ref[...] = _dot(_ln(y), wg_ref[...])


def _make_tc_d(stage, with_a):
    in_specs = [_rows(BN, D), _rows(BN, D)]
    if stage != 0:
        in_specs.append(_rows(BN, D))
    in_specs += [pl.BlockSpec((NC, BN, D), lambda i: (0, i, 0)),
                 _rows(BN, 1), _rows(BN, D), _rows(BN, 1),
                 _full((D, D)), _full((1, D))]
    n_out = 2
    if with_a:
        in_specs += [_full((D, D)), _full((D, D)), _full((D, D)),
                     _full((1, D))]
        n_out = 5
    return pl.pallas_call(
        functools.partial(_tc_d_body, stage, with_a),
        grid=(NB,),
        in_specs=in_specs,
        out_specs=[_rows(BN, D)] * n_out,
        out_shape=[jax.ShapeDtypeStruct((N, D), f32)] * n_out,
    )


_tc_d = {(s, wa): _make_tc_d(s, wa)
         for s in range(4) for wa in (True, False)}


def _tc_out_pre_body(sol_ref, dinv_ref, ts_ref):
    ts_ref[...] = dinv_ref[...] * _silu(sol_ref[...])


_tc_out_pre = pl.pallas_call(
    _tc_out_pre_body,
    grid=(NB,),
    in_specs=[_rows(BN, D), _rows(BN, 1)],
    out_specs=_rows(BN, D),
    out_shape=jax.ShapeDtypeStruct((N, D), f32),
)


def _tc_out_post_body(aggp_ref, ts_ref, wout_ref, dinv_ref, bout_ref, y_ref):
    aggp = aggp_ref[...]
    t = aggp[0] + aggp[1] + ts_ref[...]
    y_ref[...] = dinv_ref[...] * _dot(t, wout_ref[...]) + bout_ref[...]


_tc_out_post = pl.pallas_call(
    _tc_out_post_body,
    grid=(NB,),
    in_specs=[pl.BlockSpec((NC, BN, D), lambda i: (0, i, 0)),
              _rows(BN, D), _full((D, D_IN)), _rows(BN, 1), _full((1, D_IN))],
    out_specs=_rows(BN, D_IN),
    out_shape=jax.ShapeDtypeStruct((N, D_IN), f32),
)


def kernel(node_embeddings, edge_index, W_in, b_in, W_out, b_out, W_g, b_g,
           W_e1, b_e1, W_e2, b_e2, W_e3, b_e3, alpha, w_mat, d_vec):
    src = edge_index[0]
    dst = edge_index[1]
    w1a = W_e1[:D]
    w1b = W_e1[D:]
    be1 = b_e1.reshape(1, D)
    be2 = b_e2.reshape(1, D_IN)
    w3 = W_e3.reshape(1, D_IN)
    bg = b_g.reshape(1, D)
    bi = b_in.reshape(1, D)
    bo = b_out.reshape(1, D_IN)
    al = alpha.reshape(N, 1)
    dv = d_vec.reshape(1, D)

    w2 = _tc_w2(w_mat, dv)
    deg0p = _sc_count(dst).reshape(NC, N_PAD, L)
    xws0, dinv0 = _tc_din(node_embeddings, W_in, deg0p)
    agg0p = _sc_msg_plain(xws0, src, dst).reshape(NC, N_PAD, D)
    x0 = _tc_in_post(agg0p, xws0, dinv0, bi)

    xa, xb, xw = _tc_a(x0, w1a, w1b, W_g, be1)
    sols = []
    cur = x0
    z = x0
    acc = None
    for step in range(2):
        for stage in range(4):
            hsum = _sc_gather_add(xa, xb, src, dst)
            ev_rows = _tc_edge(hsum, W_e2, be2, w3, b_e3)
            degp = _sc_deg_w(ev_rows, dst).reshape(NC, N_PAD, L)
            dinv, xws = _tc_c(degp, xw)
            aggp = _sc_msg_ev(xws, ev_rows, src, dst).reshape(NC, N_PAD, D)
            with_a = not (step == 1 and stage == 3)
            args = [z, cur] + ([] if stage == 0 else [acc]) + \
                [aggp, dinv, xws, al, w2, bg]
            if with_a:
                args += [w1a, w1b, W_g, be1]
                y, acc, xa, xb, xw = _tc_d[(stage, True)](*args)
            else:
                y, acc = _tc_d[(stage, False)](*args)
            z = y
            if stage == 3:
                cur = y
        sols.append(cur)

    outs = []
    for i in range(2):
        ts = _tc_out_pre(sols[i], dinv0)
        aggo = _sc_msg_plain(ts, src, dst).reshape(NC, N_PAD, D)
        outs.append(_tc_out_post(aggo, ts, W_out, dinv0, bo))
    return (jnp.stack(outs, axis=0), outs[-1])


# speedup vs baseline: 9.0574x; 1.3039x over previous
import functools

import jax
import jax.numpy as jnp
from jax import lax
from jax.experimental import pallas as pl
from jax.experimental.pallas import tpu as pltpu
from jax.experimental.pallas import tpu_sc as plsc

N = 10000
E = 160000
D = 128
D_IN = 64
K = 128
NCHUNKS = E // K
NC = 2
NS = 16
NW = NC * NS
N_PAD = 10112
NT = N_PAD // NS
L = 16
DT = 0.5

f32 = jnp.float32
i32 = jnp.int32

BN = 400
NB = N // BN
EB = 4000
NEB = E // EB


def _mesh():
    return plsc.VectorSubcoreMesh(core_axis_name="c", subcore_axis_name="s")


def _wid():
    cid = lax.axis_index("c")
    sid = lax.axis_index("s")
    return cid, sid, sid * NC + cid


def _zero_vmem2d(buf, nrows, ncols):
    zv = jnp.zeros((L,), f32)

    def row(i, c):
        for g in range(ncols // L):
            buf[i, pl.ds(g * L, L)] = zv
        return c

    lax.fori_loop(0, nrows, row, 0)


def _zero_shared_slice(accum, zbuf, sid, zrows):
    base = sid * NT
    off = 0
    while off < NT:
        n = min(zrows, NT - off)
        pltpu.sync_copy(zbuf.at[pl.ds(0, n)], accum.at[pl.ds(base + off, n)])
        off += n


EPW = E // NW
SSZ = 40
NSROW = EPW // SSZ
SROWP = 128


def _seg_pipeline(fire, compute, store, spg):
    ind, outd = {}, {}
    ind[0] = fire(0, 0)
    if spg > 1:
        ind[1] = fire(1, 1)
    for k in range(spg):
        b = k % 3
        for d in ind[k]:
            d.wait()
        compute(k, b)
        outd[k] = store(k, b)
        if k >= 1:
            for d in outd[k - 1]:
                d.wait()
        if k + 2 < spg:
            ind[k + 2] = fire(k + 2, (k + 2) % 3)
    for d in outd[spg - 1]:
        d.wait()


def _zero_accum(buf, accum, sid, dw):
    bs = buf.shape[0]
    _zero_vmem2d(buf, bs, dw)
    base = sid * NT
    off = 0
    while off < NT:
        n = min(bs, NT - off)
        pltpu.sync_copy(buf.at[pl.ds(0, n)], accum.at[pl.ds(base + off, n)])
        off += n
    plsc.subcore_barrier()


AGSZ = 128
ASPG = 8
ANSEG = 4
_ATAIL = ((128, 128, 128, 128, 128, 128, 128, 8))


def _sc_gather_add_body(xa, xb, src, dst, out,
                        s1d, d1d, ra0, ra1, ra2, rb0, rb1, rb2,
                        gs0, gs1, gs2, ss0, ss1, ss2):
    cid, sid, wid = _wid()
    ebase = wid * EPW
    pltpu.sync_copy(src.at[pl.ds(ebase, EPW)], s1d)
    pltpu.sync_copy(dst.at[pl.ds(ebase, EPW)], d1d)
    ras, rbs = (ra0, ra1, ra2), (rb0, rb1, rb2)
    gsems, ssems = (gs0, gs1, gs2), (ss0, ss1, ss2)

    def run_segment(soff, sizes):
        def fire(k, b):
            off = soff + k * AGSZ
            n = sizes[k]
            return [
                pltpu.async_copy(xa.at[s1d.at[pl.ds(off, n)]],
                                 ras[b].at[pl.ds(0, n)], gsems[b]),
                pltpu.async_copy(xb.at[d1d.at[pl.ds(off, n)]],
                                 rbs[b].at[pl.ds(0, n)], gsems[b]),
            ]

        def compute(k, b):
            def row(i, c2):
                for gg in range(D // L):
                    s = pl.ds(gg * L, L)
                    ras[b][i, s] = ras[b][i, s] + rbs[b][i, s]
                return c2

            lax.fori_loop(0, sizes[k], row, 0)

        def store(k, b):
            off = soff + k * AGSZ
            n = sizes[k]
            return [pltpu.async_copy(ras[b].at[pl.ds(0, n)],
                                     out.at[pl.ds(ebase + off, n)], ssems[b])]

        _seg_pipeline(fire, compute, store, len(sizes))

    def seg(s, c):
        run_segment(s * (ASPG * AGSZ), (AGSZ,) * ASPG)
        return c

    lax.fori_loop(0, ANSEG, seg, 0)
    run_segment(ANSEG * ASPG * AGSZ, _ATAIL)


_sc_gather_add = functools.partial(
    pl.kernel,
    out_type=jax.ShapeDtypeStruct((E, D), f32),
    mesh=_mesh(),
    scratch_types=(
        [pltpu.VMEM((EPW,), i32)] * 2
        + [pltpu.VMEM((AGSZ, D), f32)] * 6
        + [pltpu.SemaphoreType.DMA] * 6
    ),
)(_sc_gather_add_body)


MGSZ = 40
MSPG = 5
MNSEG = EPW // (MGSZ * MSPG)


def _sc_msg_body(use_ev, *refs):
    if use_ev:
        (table, ev_rows, src, d2h, out, ix0, ix1, ix2, d2, r0, r1, r2,
         e0, e1, e2, g0, g1, g2, s0, s1, s2, accum) = refs
        evb = (e0, e1, e2)
    else:
        (table, src, d2h, out, ix0, ix1, ix2, d2, r0, r1, r2,
         g0, g1, g2, s0, s1, s2, accum) = refs
    rows, ixs = (r0, r1, r2), (ix0, ix1, ix2)
    gsems, ssems = (g0, g1, g2), (s0, s1, s2)
    cid, sid, wid = _wid()
    ebase = wid * EPW
    pltpu.sync_copy(d2h.at[pl.ds(wid * SROWP, SROWP)], d2)
    _zero_accum(r0, accum, sid, D)

    def seg(s, c):
        sbase = s * (MGSZ * MSPG)

        def fire(k, b):
            off = ebase + sbase + k * MGSZ
            pltpu.sync_copy(src.at[pl.ds(off, MGSZ)], ixs[b])
            ds = [pltpu.async_copy(table.at[ixs[b]], rows[b], gsems[b])]
            if use_ev:
                ds.append(pltpu.async_copy(ev_rows.at[pl.ds(off, MGSZ)],
                                           evb[b], gsems[b]))
            return ds

        def compute(k, b):
            if use_ev:
                def row(i, c2):
                    sv = evb[b][i, :]
                    for gg in range(D // L):
                        sl = pl.ds(gg * L, L)
                        rows[b][i, sl] = rows[b][i, sl] * sv
                    return c2

                lax.fori_loop(0, MGSZ, row, 0)

        def store(k, b):
            r = s * MSPG + k
            return [pltpu.async_copy(rows[b], accum.at[d2.at[r]],
                                     ssems[b], add=True)]

        _seg_pipeline(fire, compute, store, MSPG)
        return c

    lax.fori_loop(0, MNSEG, seg, 0)
    plsc.subcore_barrier()
    pltpu.sync_copy(accum.at[pl.ds(sid * NT, NT)],
                    out.at[pl.ds(cid * N_PAD + sid * NT, NT)])


def _make_sc_msg(use_ev):
    scratch = ([pltpu.VMEM((MGSZ,), i32)] * 3
               + [pltpu.VMEM((SROWP, SSZ), i32)]
               + [pltpu.VMEM((MGSZ, D), f32)] * 3)
    if use_ev:
        scratch += [pltpu.VMEM((MGSZ, L), f32)] * 3
    scratch += ([pltpu.SemaphoreType.DMA] * 6
                + [pltpu.VMEM_SHARED((N_PAD, D), f32)])
    return functools.partial(
        pl.kernel,
        out_type=jax.ShapeDtypeStruct((NC * N_PAD, D), f32),
        mesh=_mesh(),
        scratch_types=scratch,
    )(functools.partial(_sc_msg_body, use_ev))


_sc_msg_ev = _make_sc_msg(True)
_sc_msg_plain = _make_sc_msg(False)


def _sc_deg_body(weighted, *refs):
    if weighted:
        w_rows, d2h, out, d2, e0, e1, e2, g0, g1, g2, s0, s1, s2, accum = refs
        bufs, gsems = (e0, e1, e2), (g0, g1, g2)
    else:
        d2h, out, d2, e0, s0, s1, s2, accum = refs
        bufs = (e0, e0, e0)
    ssems = (s0, s1, s2)
    cid, sid, wid = _wid()
    ebase = wid * EPW
    pltpu.sync_copy(d2h.at[pl.ds(wid * SROWP, SROWP)], d2)
    _zero_accum(e0, accum, sid, L)
    if not weighted:
        ones = jnp.ones((L,), f32)

        def orow(i, c):
            e0[i, :] = ones
            return c

        lax.fori_loop(0, MGSZ, orow, 0)

    def seg(s, c):
        sbase = ebase + s * (MGSZ * MSPG)

        def fire(k, b):
            if not weighted:
                return []
            return [pltpu.async_copy(w_rows.at[pl.ds(sbase + k * MGSZ, MGSZ)],
                                     bufs[b], gsems[b])]

        def store(k, b):
            r = s * MSPG + k
            return [pltpu.async_copy(bufs[b], accum.at[d2.at[r]],
                                     ssems[b], add=True)]

        _seg_pipeline(fire, lambda k, b: None, store, MSPG)
        return c

    lax.fori_loop(0, MNSEG, seg, 0)
    plsc.subcore_barrier()
    pltpu.sync_copy(accum.at[pl.ds(sid * NT, NT)],
                    out.at[pl.ds(cid * N_PAD + sid * NT, NT)])


def _make_sc_deg(weighted):
    scratch = [pltpu.VMEM((SROWP, SSZ), i32)]
    if weighted:
        scratch += ([pltpu.VMEM((MGSZ, L), f32)] * 3
                    + [pltpu.SemaphoreType.DMA] * 3)
    else:
        scratch += [pltpu.VMEM((MGSZ, L), f32)]
    scratch += ([pltpu.SemaphoreType.DMA] * 3
                + [pltpu.VMEM_SHARED((N_PAD, L), f32)])
    return functools.partial(
        pl.kernel,
        out_type=jax.ShapeDtypeStruct((NC * N_PAD, L), f32),
        mesh=_mesh(),
        scratch_types=scratch,
    )(functools.partial(_sc_deg_body, weighted))


_sc_deg_w = _make_sc_deg(True)
_sc_count = _make_sc_deg(False)


def _full(shape):
    return pl.BlockSpec(shape, lambda i: tuple(0 for _ in shape))


def _rows(b, width):
    return pl.BlockSpec((b, width), lambda i: (i, 0))


def _ln(z):
    mu = jnp.mean(z, axis=-1, keepdims=True)
    zc = z - mu
    var = jnp.mean(zc * zc, axis=-1, keepdims=True)
    return zc * lax.rsqrt(var + 1e-5)


def _silu(x):
    return x * jax.nn.sigmoid(x)


def _dot(a, b):
    return jnp.dot(a, b, preferred_element_type=f32)


def _tc_w2_body(wm_ref, dv_ref, out_ref):
    wm = wm_ref[...]
    dcl = jnp.clip(dv_ref[...], 0.0, 1.0)
    out_ref[...] = lax.dot_general(wm * dcl, wm, (((1,), (1,)), ((), ())),
                                   preferred_element_type=f32)


_tc_w2 = pl.pallas_call(
    _tc_w2_body,
    grid=(1,),
    in_specs=[_full((D, D)), _full((1, D))],
    out_specs=_full((D, D)),
    out_shape=jax.ShapeDtypeStruct((D, D), f32),
)


def _tc_a_body(z_ref, w1a_ref, w1b_ref, wg_ref, be1_ref, xa_ref, xb_ref, xw_ref):
    z = z_ref[...]
    xa_ref[...] = _dot(z, w1a_ref[...]) + be1_ref[...]
    xb_ref[...] = _dot(z, w1b_ref[...])
    xw_ref[...] = _dot(_ln(z), wg_ref[...])


_tc_a = pl.pallas_call(
    _tc_a_body,
    grid=(NB,),
    in_specs=[_rows(BN, D), _full((D, D)), _full((D, D)), _full((D, D)),
              _full((1, D))],
    out_specs=[_rows(BN, D)] * 3,
    out_shape=[jax.ShapeDtypeStruct((N, D), f32)] * 3,
)


def _tc_edge_body(hs_ref, we2_ref, be2_ref, w3_ref, be3_ref, ev_ref):
    h = _silu(hs_ref[...])
    u = _dot(h, we2_ref[...]) + be2_ref[...]
    u = _silu(u)
    t = jnp.sum(u * w3_ref[...], axis=-1, keepdims=True) + be3_ref[0]
    ev_ref[...] = jnp.broadcast_to(jax.nn.sigmoid(t), (t.shape[0], L))


_tc_edge = pl.pallas_call(
    _tc_edge_body,
    grid=(NEB,),
    in_specs=[_rows(EB, D), _full((D, D_IN)), _full((1, D_IN)),
              _full((1, D_IN)),
              pl.BlockSpec(memory_space=pltpu.SMEM)],
    out_specs=_rows(EB, L),
    out_shape=jax.ShapeDtypeStruct((E, L), f32),
)


def _deg_to_dinv(degp):
    deg = 1.0 + degp[0, :, 0:1] + degp[1, :, 0:1]
    return lax.rsqrt(deg)


def _tc_din_body(emb_ref, win_ref, degp_ref, xws_ref, dinv_ref):
    dinv = _deg_to_dinv(degp_ref[...])
    xws_ref[...] = dinv * _dot(emb_ref[...], win_ref[...])
    dinv_ref[...] = dinv


_tc_din = pl.pallas_call(
    _tc_din_body,
    grid=(NB,),
    in_specs=[_rows(BN, D_IN), _full((D_IN, D)),
              pl.BlockSpec((NC, BN, L), lambda i: (0, i, 0))],
    out_specs=[_rows(BN, D), _rows(BN, 1)],
    out_shape=[jax.ShapeDtypeStruct((N, D), f32),
               jax.ShapeDtypeStruct((N, 1), f32)],
)


def _tc_in_post_body(aggp_ref, xws_ref, dinv_ref, bin_ref, x0_ref):
    aggp = aggp_ref[...]
    t = dinv_ref[...] * (aggp[0] + aggp[1] + xws_ref[...]) + bin_ref[...]
    nrm = jnp.sqrt(jnp.sum(t * t, axis=-1, keepdims=True))
    x0_ref[...] = t / jnp.maximum(nrm, 1e-12)


_tc_in_post = pl.pallas_call(
    _tc_in_post_body,
    grid=(NB,),
    in_specs=[pl.BlockSpec((NC, BN, D), lambda i: (0, i, 0)),
              _rows(BN, D), _rows(BN, 1), _full((1, D))],
    out_specs=_rows(BN, D),
    out_shape=jax.ShapeDtypeStruct((N, D), f32),
)


def _tc_c_body(degp_ref, xw_ref, dinv_ref, xws_ref):
    dinv = _deg_to_dinv(degp_ref[...])
    dinv_ref[...] = dinv
    xws_ref[...] = dinv * xw_ref[...]


_tc_c = pl.pallas_call(
    _tc_c_body,
    grid=(NB,),
    in_specs=[pl.BlockSpec((NC, BN, L), lambda i: (0, i, 0)), _rows(BN, D)],
    out_specs=[_rows(BN, 1), _rows(BN, D)],
    out_shape=[jax.ShapeDtypeStruct((N, 1), f32),
               jax.ShapeDtypeStruct((N, D), f32)],
)


def _tc_d_body(stage, with_a, *refs):
    if stage == 0:
        (z_ref, cur_ref, aggp_ref, dinv_ref, xws_ref, al_ref, w2_ref, bg_ref,
         *rest) = refs
        acc_ref = None
    else:
        (z_ref, cur_ref, acc_ref, aggp_ref, dinv_ref, xws_ref, al_ref, w2_ref,
         bg_ref, *rest) = refs
    if with_a:
        w1a_ref, w1b_ref, wg_ref, be1_ref = rest[:4]
        outs = rest[4:]
        y_ref, accn_ref, xa_ref, xb_ref, xw_ref = outs
    else:
        y_ref, accn_ref = rest

    z = z_ref[...]
    aggp = aggp_ref[...]
    gn = dinv_ref[...] * (aggp[0] + aggp[1] + xws_ref[...]) + bg_ref[...] \
        - _ln(z)
    a2 = jax.nn.sigmoid(al_ref[...]) * 0.5
    k = a2 * gn - 2.0 * z + _dot(z, w2_ref[...])
    if stage == 0:
        accn = k
    elif stage in (1, 2):
        accn = acc_ref[...] + 2.0 * k
    else:
        accn = acc_ref[...] + k
    accn_ref[...] = accn
    cur = cur_ref[...]
    if stage < 3:
        y = cur + (DT / 2.0 if stage < 2 else DT) * k
    else:
        y = cur + (DT / 6.0) * accn
    y_ref[...] = y
    if with_a:
        xa_ref[...] = _dot(y, w1a_ref[...]) + be1_ref[...]
        xb_ref[...] = _dot(y, w1b_ref[...])
        xw_ref[...] = _dot(_ln(y), wg_ref[...])


def _make_tc_d(stage, with_a):
    in_specs = [_rows(BN, D), _rows(BN, D)]
    if stage != 0:
        in_specs.append(_rows(BN, D))
    in_specs += [pl.BlockSpec((NC, BN, D), lambda i: (0, i, 0)),
                 _rows(BN, 1), _rows(BN, D), _rows(BN, 1),
                 _full((D, D)), _full((1, D))]
    n_out = 2
    if with_a:
        in_specs += [_full((D, D)), _full((D, D)), _full((D, D)),
                     _full((1, D))]
        n_out = 5
    return pl.pallas_call(
        functools.partial(_tc_d_body, stage, with_a),
        grid=(NB,),
        in_specs=in_specs,
        out_specs=[_rows(BN, D)] * n_out,
        out_shape=[jax.ShapeDtypeStruct((N, D), f32)] * n_out,
    )


_tc_d = {(s, wa): _make_tc_d(s, wa)
         for s in range(4) for wa in (True, False)}


def _tc_out_pre_body(sol_ref, dinv_ref, ts_ref):
    ts_ref[...] = dinv_ref[...] * _silu(sol_ref[...])


_tc_out_pre = pl.pallas_call(
    _tc_out_pre_body,
    grid=(NB,),
    in_specs=[_rows(BN, D), _rows(BN, 1)],
    out_specs=_rows(BN, D),
    out_shape=jax.ShapeDtypeStruct((N, D), f32),
)


def _tc_out_post_body(aggp_ref, ts_ref, wout_ref, dinv_ref, bout_ref, y_ref):
    aggp = aggp_ref[...]
    t = aggp[0] + aggp[1] + ts_ref[...]
    y_ref[...] = dinv_ref[...] * _dot(t, wout_ref[...]) + bout_ref[...]


_tc_out_post = pl.pallas_call(
    _tc_out_post_body,
    grid=(NB,),
    in_specs=[pl.BlockSpec((NC, BN, D), lambda i: (0, i, 0)),
              _rows(BN, D), _full((D, D_IN)), _rows(BN, 1), _full((1, D_IN))],
    out_specs=_rows(BN, D_IN),
    out_shape=jax.ShapeDtypeStruct((N, D_IN), f32),
)


def kernel(node_embeddings, edge_index, W_in, b_in, W_out, b_out, W_g, b_g,
           W_e1, b_e1, W_e2, b_e2, W_e3, b_e3, alpha, w_mat, d_vec):
    src = edge_index[0]
    dst = edge_index[1]
    d2h = jnp.pad(dst.reshape(NW, NSROW, SSZ),
                  ((0, 0), (0, SROWP - NSROW), (0, 0))).reshape(NW * SROWP, SSZ)
    w1a = W_e1[:D]
    w1b = W_e1[D:]
    be1 = b_e1.reshape(1, D)
    be2 = b_e2.reshape(1, D_IN)
    w3 = W_e3.reshape(1, D_IN)
    bg = b_g.reshape(1, D)
    bi = b_in.reshape(1, D)
    bo = b_out.reshape(1, D_IN)
    al = alpha.reshape(N, 1)
    dv = d_vec.reshape(1, D)

    w2 = _tc_w2(w_mat, dv)
    deg0p = _sc_count(d2h).reshape(NC, N_PAD, L)
    xws0, dinv0 = _tc_din(node_embeddings, W_in, deg0p)
    agg0p = _sc_msg_plain(xws0, src, d2h).reshape(NC, N_PAD, D)
    x0 = _tc_in_post(agg0p, xws0, dinv0, bi)

    xa, xb, xw = _tc_a(x0, w1a, w1b, W_g, be1)
    sols = []
    cur = x0
    z = x0
    acc = None
    for step in range(2):
        for stage in range(4):
            hsum = _sc_gather_add(xa, xb, src, dst)
            ev_rows = _tc_edge(hsum, W_e2, be2, w3, b_e3)
            degp = _sc_deg_w(ev_rows, d2h).reshape(NC, N_PAD, L)
            dinv, xws = _tc_c(degp, xw)
            aggp = _sc_msg_ev(xws, ev_rows, src, d2h).reshape(NC, N_PAD, D)
            with_a = not (step == 1 and stage == 3)
            args = [z, cur] + ([] if stage == 0 else [acc]) + \
                [aggp, dinv, xws, al, w2, bg]
            if with_a:
                args += [w1a, w1b, W_g, be1]
                y, acc, xa, xb, xw = _tc_d[(stage, True)](*args)
            else:
                y, acc = _tc_d[(stage, False)](*args)
            z = y
            if stage == 3:
                cur = y
        sols.append(cur)

    outs = []
    for i in range(2):
        ts = _tc_out_pre(sols[i], dinv0)
        aggo = _sc_msg_plain(ts, src, d2h).reshape(NC, N_PAD, D)
        outs.append(_tc_out_post(aggo, ts, W_out, dinv0, bo))
    return (jnp.stack(outs, axis=0), outs[-1])


# speedup vs baseline: 9.3687x; 1.0344x over previous
import functools

import jax
import jax.numpy as jnp
from jax import lax
from jax.experimental import pallas as pl
from jax.experimental.pallas import tpu as pltpu
from jax.experimental.pallas import tpu_sc as plsc

N = 10000
E = 160000
D = 128
D_IN = 64
K = 128
NCHUNKS = E // K
NC = 2
NS = 16
NW = NC * NS
N_PAD = 10112
NT = N_PAD // NS
L = 16
DT = 0.5

f32 = jnp.float32
i32 = jnp.int32

BN = 400
NB = N // BN
EB = 4000
NEB = E // EB


def _mesh():
    return plsc.VectorSubcoreMesh(core_axis_name="c", subcore_axis_name="s")


def _wid():
    cid = lax.axis_index("c")
    sid = lax.axis_index("s")
    return cid, sid, sid * NC + cid


def _zero_vmem2d(buf, nrows, ncols):
    zv = jnp.zeros((L,), f32)

    def row(i, c):
        for g in range(ncols // L):
            buf[i, pl.ds(g * L, L)] = zv
        return c

    lax.fori_loop(0, nrows, row, 0)


def _zero_shared_slice(accum, zbuf, sid, zrows):
    base = sid * NT
    off = 0
    while off < NT:
        n = min(zrows, NT - off)
        pltpu.sync_copy(zbuf.at[pl.ds(0, n)], accum.at[pl.ds(base + off, n)])
        off += n


EPW = E // NW
SSZ = 40
NSROW = EPW // SSZ
SROWP = 128


def _seg_pipeline(fire, compute, store, spg):
    ind, outd = {}, {}
    ind[0] = fire(0, 0)
    if spg > 1:
        ind[1] = fire(1, 1)
    for k in range(spg):
        b = k % 3
        for d in ind[k]:
            d.wait()
        compute(k, b)
        outd[k] = store(k, b)
        if k >= 1:
            for d in outd[k - 1]:
                d.wait()
        if k + 2 < spg:
            ind[k + 2] = fire(k + 2, (k + 2) % 3)
    for d in outd[spg - 1]:
        d.wait()


def _zero_accum(buf, accum, sid, dw):
    bs = buf.shape[0]
    _zero_vmem2d(buf, bs, dw)
    base = sid * NT
    off = 0
    while off < NT:
        n = min(bs, NT - off)
        pltpu.sync_copy(buf.at[pl.ds(0, n)], accum.at[pl.ds(base + off, n)])
        off += n
    plsc.subcore_barrier()


AGSZ = 128
ASPG = 8
ANSEG = 4
_ATAIL = ((128, 128, 128, 128, 128, 128, 128, 8))


def _sc_gather_add_body(xa, xb, src, dst, out,
                        s1d, d1d, ra0, ra1, ra2, rb0, rb1, rb2,
                        gs0, gs1, gs2, ss0, ss1, ss2):
    cid, sid, wid = _wid()
    ebase = wid * EPW
    pltpu.sync_copy(src.at[pl.ds(ebase, EPW)], s1d)
    pltpu.sync_copy(dst.at[pl.ds(ebase, EPW)], d1d)
    ras, rbs = (ra0, ra1, ra2), (rb0, rb1, rb2)
    gsems, ssems = (gs0, gs1, gs2), (ss0, ss1, ss2)

    def run_segment(soff, sizes):
        def fire(k, b):
            off = soff + k * AGSZ
            n = sizes[k]
            return [
                pltpu.async_copy(xa.at[s1d.at[pl.ds(off, n)]],
                                 ras[b].at[pl.ds(0, n)], gsems[b]),
                pltpu.async_copy(xb.at[d1d.at[pl.ds(off, n)]],
                                 rbs[b].at[pl.ds(0, n)], gsems[b]),
            ]

        def compute(k, b):
            def row(i, c2):
                for gg in range(D // L):
                    s = pl.ds(gg * L, L)
                    ras[b][i, s] = ras[b][i, s] + rbs[b][i, s]
                return c2

            lax.fori_loop(0, sizes[k], row, 0)

        def store(k, b):
            off = soff + k * AGSZ
            n = sizes[k]
            return [pltpu.async_copy(ras[b].at[pl.ds(0, n)],
                                     out.at[pl.ds(ebase + off, n)], ssems[b])]

        _seg_pipeline(fire, compute, store, len(sizes))

    def seg(s, c):
        run_segment(s * (ASPG * AGSZ), (AGSZ,) * ASPG)
        return c

    lax.fori_loop(0, ANSEG, seg, 0)
    run_segment(ANSEG * ASPG * AGSZ, _ATAIL)


_sc_gather_add = functools.partial(
    pl.kernel,
    out_type=jax.ShapeDtypeStruct((E, D), f32),
    mesh=_mesh(),
    scratch_types=(
        [pltpu.VMEM((EPW,), i32)] * 2
        + [pltpu.VMEM((AGSZ, D), f32)] * 6
        + [pltpu.SemaphoreType.DMA] * 6
    ),
)(_sc_gather_add_body)


MGSZ = 40
MSPG = 5
MNSEG = EPW // (MGSZ * MSPG)


def _sc_msg_body(use_ev, *refs):
    if use_ev:
        (table, ev_rows, src, d2h, out, ix0, ix1, ix2, ix3, ix4,
         d2, r0, r1, r2, e0, e1, e2,
         i0, i1, i2, i3, i4, g0, g1, g2, s0, s1, s2, accum) = refs
        evb = (e0, e1, e2)
    else:
        (table, src, d2h, out, ix0, ix1, ix2, ix3, ix4,
         d2, r0, r1, r2,
         i0, i1, i2, i3, i4, g0, g1, g2, s0, s1, s2, accum) = refs
    rows, ixs = (r0, r1, r2), (ix0, ix1, ix2, ix3, ix4)
    isems = (i0, i1, i2, i3, i4)
    gsems, ssems = (g0, g1, g2), (s0, s1, s2)
    cid, sid, wid = _wid()
    ebase = wid * EPW
    pltpu.sync_copy(d2h.at[pl.ds(wid * SROWP, SROWP)], d2)
    _zero_accum(r0, accum, sid, D)

    def seg(s, c):
        sbase = s * (MGSZ * MSPG)

        def idx_fetch(k):
            off = ebase + sbase + k * MGSZ
            return pltpu.async_copy(src.at[pl.ds(off, MGSZ)], ixs[k],
                                    isems[k])

        def fire(k, b, idxd):
            off = ebase + sbase + k * MGSZ
            idxd[k].wait()
            ds = [pltpu.async_copy(table.at[ixs[k]], rows[b], gsems[b])]
            if use_ev:
                ds.append(pltpu.async_copy(ev_rows.at[pl.ds(off, MGSZ)],
                                           evb[b], gsems[b]))
            if k + 2 < MSPG:
                idxd[k + 2] = idx_fetch(k + 2)
            return ds

        def compute(k, b):
            if use_ev:
                def row(i, c2):
                    sv = evb[b][i, :]
                    for gg in range(D // L):
                        sl = pl.ds(gg * L, L)
                        rows[b][i, sl] = rows[b][i, sl] * sv
                    return c2

                lax.fori_loop(0, MGSZ, row, 0)

        def store(k, b):
            r = s * MSPG + k
            return [pltpu.async_copy(rows[b], accum.at[d2.at[r]],
                                     ssems[b], add=True)]

        idxd = {0: idx_fetch(0), 1: idx_fetch(1)}
        _seg_pipeline(lambda k, b: fire(k, b, idxd), compute, store, MSPG)
        return c

    lax.fori_loop(0, MNSEG, seg, 0)
    plsc.subcore_barrier()
    pltpu.sync_copy(accum.at[pl.ds(sid * NT, NT)],
                    out.at[pl.ds(cid * N_PAD + sid * NT, NT)])


def _make_sc_msg(use_ev):
    scratch = ([pltpu.VMEM((MGSZ,), i32)] * 5
               + [pltpu.VMEM((SROWP, SSZ), i32)]
               + [pltpu.VMEM((MGSZ, D), f32)] * 3)
    if use_ev:
        scratch += [pltpu.VMEM((MGSZ, L), f32)] * 3
    scratch += ([pltpu.SemaphoreType.DMA] * 11
                + [pltpu.VMEM_SHARED((N_PAD, D), f32)])
    return functools.partial(
        pl.kernel,
        out_type=jax.ShapeDtypeStruct((NC * N_PAD, D), f32),
        mesh=_mesh(),
        scratch_types=scratch,
    )(functools.partial(_sc_msg_body, use_ev))


_sc_msg_ev = _make_sc_msg(True)
_sc_msg_plain = _make_sc_msg(False)


def _sc_deg_body(weighted, *refs):
    if weighted:
        w_rows, d2h, out, d2, e0, e1, e2, g0, g1, g2, s0, s1, s2, accum = refs
        bufs, gsems = (e0, e1, e2), (g0, g1, g2)
    else:
        d2h, out, d2, e0, s0, s1, s2, accum = refs
        bufs = (e0, e0, e0)
    ssems = (s0, s1, s2)
    cid, sid, wid = _wid()
    ebase = wid * EPW
    pltpu.sync_copy(d2h.at[pl.ds(wid * SROWP, SROWP)], d2)
    _zero_accum(e0, accum, sid, L)
    if not weighted:
        ones = jnp.ones((L,), f32)

        def orow(i, c):
            e0[i, :] = ones
            return c

        lax.fori_loop(0, MGSZ, orow, 0)

    def seg(s, c):
        sbase = ebase + s * (MGSZ * MSPG)

        def fire(k, b):
            if not weighted:
                return []
            return [pltpu.async_copy(w_rows.at[pl.ds(sbase + k * MGSZ, MGSZ)],
                                     bufs[b], gsems[b])]

        def store(k, b):
            r = s * MSPG + k
            return [pltpu.async_copy(bufs[b], accum.at[d2.at[r]],
                                     ssems[b], add=True)]

        _seg_pipeline(fire, lambda k, b: None, store, MSPG)
        return c

    lax.fori_loop(0, MNSEG, seg, 0)
    plsc.subcore_barrier()
    pltpu.sync_copy(accum.at[pl.ds(sid * NT, NT)],
                    out.at[pl.ds(cid * N_PAD + sid * NT, NT)])


def _make_sc_deg(weighted):
    scratch = [pltpu.VMEM((SROWP, SSZ), i32)]
    if weighted:
        scratch += ([pltpu.VMEM((MGSZ, L), f32)] * 3
                    + [pltpu.SemaphoreType.DMA] * 3)
    else:
        scratch += [pltpu.VMEM((MGSZ, L), f32)]
    scratch += ([pltpu.SemaphoreType.DMA] * 3
                + [pltpu.VMEM_SHARED((N_PAD, L), f32)])
    return functools.partial(
        pl.kernel,
        out_type=jax.ShapeDtypeStruct((NC * N_PAD, L), f32),
        mesh=_mesh(),
        scratch_types=scratch,
    )(functools.partial(_sc_deg_body, weighted))


_sc_deg_w = _make_sc_deg(True)
_sc_count = _make_sc_deg(False)


def _full(shape):
    return pl.BlockSpec(shape, lambda i: tuple(0 for _ in shape))


def _rows(b, width):
    return pl.BlockSpec((b, width), lambda i: (i, 0))


def _ln(z):
    mu = jnp.mean(z, axis=-1, keepdims=True)
    zc = z - mu
    var = jnp.mean(zc * zc, axis=-1, keepdims=True)
    return zc * lax.rsqrt(var + 1e-5)


def _silu(x):
    return x * jax.nn.sigmoid(x)


def _dot(a, b):
    return jnp.dot(a, b, preferred_element_type=f32)


def _tc_w2_body(wm_ref, dv_ref, out_ref):
    wm = wm_ref[...]
    dcl = jnp.clip(dv_ref[...], 0.0, 1.0)
    out_ref[...] = lax.dot_general(wm * dcl, wm, (((1,), (1,)), ((), ())),
                                   preferred_element_type=f32)


_tc_w2 = pl.pallas_call(
    _tc_w2_body,
    grid=(1,),
    in_specs=[_full((D, D)), _full((1, D))],
    out_specs=_full((D, D)),
    out_shape=jax.ShapeDtypeStruct((D, D), f32),
)


def _tc_a_body(z_ref, w1a_ref, w1b_ref, wg_ref, be1_ref, xa_ref, xb_ref, xw_ref):
    z = z_ref[...]
    xa_ref[...] = _dot(z, w1a_ref[...]) + be1_ref[...]
    xb_ref[...] = _dot(z, w1b_ref[...])
    xw_ref[...] = _dot(_ln(z), wg_ref[...])


_tc_a = pl.pallas_call(
    _tc_a_body,
    grid=(NB,),
    in_specs=[_rows(BN, D), _full((D, D)), _full((D, D)), _full((D, D)),
              _full((1, D))],
    out_specs=[_rows(BN, D)] * 3,
    out_shape=[jax.ShapeDtypeStruct((N, D), f32)] * 3,
)


def _tc_edge_body(hs_ref, we2_ref, be2_ref, w3_ref, be3_ref, ev_ref):
    h = _silu(hs_ref[...])
    u = _dot(h, we2_ref[...]) + be2_ref[...]
    u = _silu(u)
    t = jnp.sum(u * w3_ref[...], axis=-1, keepdims=True) + be3_ref[0]
    ev_ref[...] = jnp.broadcast_to(jax.nn.sigmoid(t), (t.shape[0], L))


_tc_edge = pl.pallas_call(
    _tc_edge_body,
    grid=(NEB,),
    in_specs=[_rows(EB, D), _full((D, D_IN)), _full((1, D_IN)),
              _full((1, D_IN)),
              pl.BlockSpec(memory_space=pltpu.SMEM)],
    out_specs=_rows(EB, L),
    out_shape=jax.ShapeDtypeStruct((E, L), f32),
)


def _deg_to_dinv(degp):
    deg = 1.0 + degp[0, :, 0:1] + degp[1, :, 0:1]
    return lax.rsqrt(deg)


def _tc_din_body(emb_ref, win_ref, degp_ref, xws_ref, dinv_ref):
    dinv = _deg_to_dinv(degp_ref[...])
    xws_ref[...] = dinv * _dot(emb_ref[...], win_ref[...])
    dinv_ref[...] = dinv


_tc_din = pl.pallas_call(
    _tc_din_body,
    grid=(NB,),
    in_specs=[_rows(BN, D_IN), _full((D_IN, D)),
              pl.BlockSpec((NC, BN, L), lambda i: (0, i, 0))],
    out_specs=[_rows(BN, D), _rows(BN, 1)],
    out_shape=[jax.ShapeDtypeStruct((N, D), f32),
               jax.ShapeDtypeStruct((N, 1), f32)],
)


def _tc_in_post_body(aggp_ref, xws_ref, dinv_ref, bin_ref, x0_ref):
    aggp = aggp_ref[...]
    t = dinv_ref[...] * (aggp[0] + aggp[1] + xws_ref[...]) + bin_ref[...]
    nrm = jnp.sqrt(jnp.sum(t * t, axis=-1, keepdims=True))
    x0_ref[...] = t / jnp.maximum(nrm, 1e-12)


_tc_in_post = pl.pallas_call(
    _tc_in_post_body,
    grid=(NB,),
    in_specs=[pl.BlockSpec((NC, BN, D), lambda i: (0, i, 0)),
              _rows(BN, D), _rows(BN, 1), _full((1, D))],
    out_specs=_rows(BN, D),
    out_shape=jax.ShapeDtypeStruct((N, D), f32),
)


def _tc_c_body(degp_ref, xw_ref, dinv_ref, xws_ref):
    dinv = _deg_to_dinv(degp_ref[...])
    dinv_ref[...] = dinv
    xws_ref[...] = dinv * xw_ref[...]


_tc_c = pl.pallas_call(
    _tc_c_body,
    grid=(NB,),
    in_specs=[pl.BlockSpec((NC, BN, L), lambda i: (0, i, 0)), _rows(BN, D)],
    out_specs=[_rows(BN, 1), _rows(BN, D)],
    out_shape=[jax.ShapeDtypeStruct((N, 1), f32),
               jax.ShapeDtypeStruct((N, D), f32)],
)


def _tc_d_body(stage, with_a, *refs):
    if stage == 0:
        (z_ref, cur_ref, aggp_ref, dinv_ref, xws_ref, al_ref, w2_ref, bg_ref,
         *rest) = refs
        acc_ref = None
    else:
        (z_ref, cur_ref, acc_ref, aggp_ref, dinv_ref, xws_ref, al_ref, w2_ref,
         bg_ref, *rest) = refs
    if with_a:
        w1a_ref, w1b_ref, wg_ref, be1_ref = rest[:4]
        outs = rest[4:]
        y_ref, accn_ref, xa_ref, xb_ref, xw_ref = outs
    else:
        y_ref, accn_ref = rest

    z = z_ref[...]
    aggp = aggp_ref[...]
    gn = dinv_ref[...] * (aggp[0] + aggp[1] + xws_ref[...]) + bg_ref[...] \
        - _ln(z)
    a2 = jax.nn.sigmoid(al_ref[...]) * 0.5
    k = a2 * gn - 2.0 * z + _dot(z, w2_ref[...])
    if stage == 0:
        accn = k
    elif stage in (1, 2):
        accn = acc_ref[...] + 2.0 * k
    else:
        accn = acc_ref[...] + k
    accn_ref[...] = accn
    cur = cur_ref[...]
    if stage < 3:
        y = cur + (DT / 2.0 if stage < 2 else DT) * k
    else:
        y = cur + (DT / 6.0) * accn
    y_ref[...] = y
    if with_a:
        xa_ref[...] = _dot(y, w1a_ref[...]) + be1_ref[...]
        xb_ref[...] = _dot(y, w1b_ref[...])
        xw_ref[...] = _dot(_ln(y), wg_ref[...])


def _make_tc_d(stage, with_a):
    in_specs = [_rows(BN, D), _rows(BN, D)]
    if stage != 0:
        in_specs.append(_rows(BN, D))
    in_specs += [pl.BlockSpec((NC, BN, D), lambda i: (0, i, 0)),
                 _rows(BN, 1), _rows(BN, D), _rows(BN, 1),
                 _full((D, D)), _full((1, D))]
    n_out = 2
    if with_a:
        in_specs += [_full((D, D)), _full((D, D)), _full((D, D)),
                     _full((1, D))]
        n_out = 5
    return pl.pallas_call(
        functools.partial(_tc_d_body, stage, with_a),
        grid=(NB,),
        in_specs=in_specs,
        out_specs=[_rows(BN, D)] * n_out,
        out_shape=[jax.ShapeDtypeStruct((N, D), f32)] * n_out,
    )


_tc_d = {(s, wa): _make_tc_d(s, wa)
         for s in range(4) for wa in (True, False)}


def _tc_out_pre_body(sol_ref, dinv_ref, ts_ref):
    ts_ref[...] = dinv_ref[...] * _silu(sol_ref[...])


_tc_out_pre = pl.pallas_call(
    _tc_out_pre_body,
    grid=(NB,),
    in_specs=[_rows(BN, D), _rows(BN, 1)],
    out_specs=_rows(BN, D),
    out_shape=jax.ShapeDtypeStruct((N, D), f32),
)


def _tc_out_post_body(aggp_ref, ts_ref, wout_ref, dinv_ref, bout_ref, y_ref):
    aggp = aggp_ref[...]
    t = aggp[0] + aggp[1] + ts_ref[...]
    y_ref[...] = dinv_ref[...] * _dot(t, wout_ref[...]) + bout_ref[...]


_tc_out_post = pl.pallas_call(
    _tc_out_post_body,
    grid=(NB,),
    in_specs=[pl.BlockSpec((NC, BN, D), lambda i: (0, i, 0)),
              _rows(BN, D), _full((D, D_IN)), _rows(BN, 1), _full((1, D_IN))],
    out_specs=_rows(BN, D_IN),
    out_shape=jax.ShapeDtypeStruct((N, D_IN), f32),
)


def kernel(node_embeddings, edge_index, W_in, b_in, W_out, b_out, W_g, b_g,
           W_e1, b_e1, W_e2, b_e2, W_e3, b_e3, alpha, w_mat, d_vec):
    src = edge_index[0]
    dst = edge_index[1]
    d2h = jnp.pad(dst.reshape(NW, NSROW, SSZ),
                  ((0, 0), (0, SROWP - NSROW), (0, 0))).reshape(NW * SROWP, SSZ)
    w1a = W_e1[:D]
    w1b = W_e1[D:]
    be1 = b_e1.reshape(1, D)
    be2 = b_e2.reshape(1, D_IN)
    w3 = W_e3.reshape(1, D_IN)
    bg = b_g.reshape(1, D)
    bi = b_in.reshape(1, D)
    bo = b_out.reshape(1, D_IN)
    al = alpha.reshape(N, 1)
    dv = d_vec.reshape(1, D)

    w2 = _tc_w2(w_mat, dv)
    deg0p = _sc_count(d2h).reshape(NC, N_PAD, L)
    xws0, dinv0 = _tc_din(node_embeddings, W_in, deg0p)
    agg0p = _sc_msg_plain(xws0, src, d2h).reshape(NC, N_PAD, D)
    x0 = _tc_in_post(agg0p, xws0, dinv0, bi)

    xa, xb, xw = _tc_a(x0, w1a, w1b, W_g, be1)
    sols = []
    cur = x0
    z = x0
    acc = None
    for step in range(2):
        for stage in range(4):
            hsum = _sc_gather_add(xa, xb, src, dst)
            ev_rows = _tc_edge(hsum, W_e2, be2, w3, b_e3)
            degp = _sc_deg_w(ev_rows, d2h).reshape(NC, N_PAD, L)
            dinv, xws = _tc_c(degp, xw)
            aggp = _sc_msg_ev(xws, ev_rows, src, d2h).reshape(NC, N_PAD, D)
            with_a = not (step == 1 and stage == 3)
            args = [z, cur] + ([] if stage == 0 else [acc]) + \
                [aggp, dinv, xws, al, w2, bg]
            if with_a:
                args += [w1a, w1b, W_g, be1]
                y, acc, xa, xb, xw = _tc_d[(stage, True)](*args)
            else:
                y, acc = _tc_d[(stage, False)](*args)
            z = y
            if stage == 3:
                cur = y
        sols.append(cur)

    outs = []
    for i in range(2):
        ts = _tc_out_pre(sols[i], dinv0)
        aggo = _sc_msg_plain(ts, src, d2h).reshape(NC, N_PAD, D)
        outs.append(_tc_out_post(aggo, ts, W_out, dinv0, bo))
    return (jnp.stack(outs, axis=0), outs[-1])


# speedup vs baseline: 9.5283x; 1.0170x over previous
import functools

import jax
import jax.numpy as jnp
from jax import lax
from jax.experimental import pallas as pl
from jax.experimental.pallas import tpu as pltpu
from jax.experimental.pallas import tpu_sc as plsc

N = 10000
E = 160000
D = 128
D_IN = 64
K = 128
NCHUNKS = E // K
NC = 2
NS = 16
NW = NC * NS
N_PAD = 10112
NT = N_PAD // NS
L = 16
DT = 0.5

f32 = jnp.float32
i32 = jnp.int32

BN = 400
NB = N // BN
EB = 4000
NEB = E // EB


def _mesh():
    return plsc.VectorSubcoreMesh(core_axis_name="c", subcore_axis_name="s")


def _wid():
    cid = lax.axis_index("c")
    sid = lax.axis_index("s")
    return cid, sid, sid * NC + cid


def _zero_vmem2d(buf, nrows, ncols):
    zv = jnp.zeros((L,), f32)

    def row(i, c):
        for g in range(ncols // L):
            buf[i, pl.ds(g * L, L)] = zv
        return c

    lax.fori_loop(0, nrows, row, 0)


def _zero_shared_slice(accum, zbuf, sid, zrows):
    base = sid * NT
    off = 0
    while off < NT:
        n = min(zrows, NT - off)
        pltpu.sync_copy(zbuf.at[pl.ds(0, n)], accum.at[pl.ds(base + off, n)])
        off += n


EPW = E // NW
SSZ = 40
NSROW = EPW // SSZ
SROWP = 128


def _seg_pipeline(fire, compute, store, spg):
    ind, outd = {}, {}
    ind[0] = fire(0, 0)
    if spg > 1:
        ind[1] = fire(1, 1)
    for k in range(spg):
        b = k % 3
        for d in ind[k]:
            d.wait()
        compute(k, b)
        outd[k] = store(k, b)
        if k >= 1:
            for d in outd[k - 1]:
                d.wait()
        if k + 2 < spg:
            ind[k + 2] = fire(k + 2, (k + 2) % 3)
    for d in outd[spg - 1]:
        d.wait()


def _zero_accum(buf, accum, sid, dw):
    bs = buf.shape[0]
    _zero_vmem2d(buf, bs, dw)
    base = sid * NT
    off = 0
    while off < NT:
        n = min(bs, NT - off)
        pltpu.sync_copy(buf.at[pl.ds(0, n)], accum.at[pl.ds(base + off, n)])
        off += n
    plsc.subcore_barrier()


AGSZ = 128
ASPG = 8
ANSEG = 4
_ATAIL = ((128, 128, 128, 128, 128, 128, 128, 8))


def _sc_gather_add_body(xa, xb, src, dst, out,
                        s1d, d1d, ra0, ra1, ra2, rb0, rb1, rb2,
                        gs0, gs1, gs2, ss0, ss1, ss2):
    cid, sid, wid = _wid()
    ebase = wid * EPW
    pltpu.sync_copy(src.at[pl.ds(ebase, EPW)], s1d)
    pltpu.sync_copy(dst.at[pl.ds(ebase, EPW)], d1d)
    ras, rbs = (ra0, ra1, ra2), (rb0, rb1, rb2)
    gsems, ssems = (gs0, gs1, gs2), (ss0, ss1, ss2)

    def run_segment(soff, sizes):
        def fire(k, b):
            off = soff + k * AGSZ
            n = sizes[k]
            return [
                pltpu.async_copy(xa.at[s1d.at[pl.ds(off, n)]],
                                 ras[b].at[pl.ds(0, n)], gsems[b]),
                pltpu.async_copy(xb.at[d1d.at[pl.ds(off, n)]],
                                 rbs[b].at[pl.ds(0, n)], gsems[b]),
            ]

        def compute(k, b):
            def row(i, c2):
                for gg in range(D // L):
                    s = pl.ds(gg * L, L)
                    ras[b][i, s] = ras[b][i, s] + rbs[b][i, s]
                return c2

            lax.fori_loop(0, sizes[k], row, 0)

        def store(k, b):
            off = soff + k * AGSZ
            n = sizes[k]
            return [pltpu.async_copy(ras[b].at[pl.ds(0, n)],
                                     out.at[pl.ds(ebase + off, n)], ssems[b])]

        _seg_pipeline(fire, compute, store, len(sizes))

    def seg(s, c):
        run_segment(s * (ASPG * AGSZ), (AGSZ,) * ASPG)
        return c

    lax.fori_loop(0, ANSEG, seg, 0)
    run_segment(ANSEG * ASPG * AGSZ, _ATAIL)


_sc_gather_add = functools.partial(
    pl.kernel,
    out_type=jax.ShapeDtypeStruct((E, D), f32),
    mesh=_mesh(),
    scratch_types=(
        [pltpu.VMEM((EPW,), i32)] * 2
        + [pltpu.VMEM((AGSZ, D), f32)] * 6
        + [pltpu.SemaphoreType.DMA] * 6
    ),
)(_sc_gather_add_body)


MGSZ = 40
MSPG = 5
MNSEG = EPW // (MGSZ * MSPG)
DSPG = 10
DNSEG = 12


def _sc_msg_body(use_ev, *refs):
    if use_ev:
        (table, ev_rows, src, d2h, out, ix0, ix1, ix2, ix3, ix4,
         d2, r0, r1, r2, e0, e1, e2,
         i0, i1, i2, i3, i4, g0, g1, g2, s0, s1, s2, accum) = refs
        evb = (e0, e1, e2)
    else:
        (table, src, d2h, out, ix0, ix1, ix2, ix3, ix4,
         d2, r0, r1, r2,
         i0, i1, i2, i3, i4, g0, g1, g2, s0, s1, s2, accum) = refs
    rows, ixs = (r0, r1, r2), (ix0, ix1, ix2, ix3, ix4)
    isems = (i0, i1, i2, i3, i4)
    gsems, ssems = (g0, g1, g2), (s0, s1, s2)
    cid, sid, wid = _wid()
    ebase = wid * EPW
    pltpu.sync_copy(d2h.at[pl.ds(wid * SROWP, SROWP)], d2)
    _zero_accum(r0, accum, sid, D)

    def seg(s, c):
        sbase = s * (MGSZ * MSPG)

        def idx_fetch(k):
            off = ebase + sbase + k * MGSZ
            return pltpu.async_copy(src.at[pl.ds(off, MGSZ)], ixs[k],
                                    isems[k])

        def fire(k, b, idxd):
            off = ebase + sbase + k * MGSZ
            idxd[k].wait()
            ds = [pltpu.async_copy(table.at[ixs[k]], rows[b], gsems[b])]
            if use_ev:
                ds.append(pltpu.async_copy(ev_rows.at[pl.ds(off, MGSZ)],
                                           evb[b], gsems[b]))
            if k + 2 < MSPG:
                idxd[k + 2] = idx_fetch(k + 2)
            return ds

        def compute(k, b):
            if use_ev:
                def row(i, c2):
                    sv = evb[b][i, :]
                    for gg in range(D // L):
                        sl = pl.ds(gg * L, L)
                        rows[b][i, sl] = rows[b][i, sl] * sv
                    return c2

                lax.fori_loop(0, MGSZ, row, 0)

        def store(k, b):
            r = s * MSPG + k
            return [pltpu.async_copy(rows[b], accum.at[d2.at[r]],
                                     ssems[b], add=True)]

        idxd = {0: idx_fetch(0), 1: idx_fetch(1)}
        _seg_pipeline(lambda k, b: fire(k, b, idxd), compute, store, MSPG)
        return c

    lax.fori_loop(0, MNSEG, seg, 0)
    plsc.subcore_barrier()
    pltpu.sync_copy(accum.at[pl.ds(sid * NT, NT)],
                    out.at[pl.ds(cid * N_PAD + sid * NT, NT)])


def _make_sc_msg(use_ev):
    scratch = ([pltpu.VMEM((MGSZ,), i32)] * 5
               + [pltpu.VMEM((SROWP, SSZ), i32)]
               + [pltpu.VMEM((MGSZ, D), f32)] * 3)
    if use_ev:
        scratch += [pltpu.VMEM((MGSZ, L), f32)] * 3
    scratch += ([pltpu.SemaphoreType.DMA] * 11
                + [pltpu.VMEM_SHARED((N_PAD, D), f32)])
    return functools.partial(
        pl.kernel,
        out_type=jax.ShapeDtypeStruct((NC * N_PAD, D), f32),
        mesh=_mesh(),
        scratch_types=scratch,
    )(functools.partial(_sc_msg_body, use_ev))


_sc_msg_ev = _make_sc_msg(True)
_sc_msg_plain = _make_sc_msg(False)


def _sc_deg_body(weighted, *refs):
    if weighted:
        w_rows, d2h, out, d2, e0, e1, e2, g0, g1, g2, s0, s1, s2, accum = refs
        bufs, gsems = (e0, e1, e2), (g0, g1, g2)
    else:
        d2h, out, d2, e0, s0, s1, s2, accum = refs
        bufs = (e0, e0, e0)
    ssems = (s0, s1, s2)
    cid, sid, wid = _wid()
    ebase = wid * EPW
    pltpu.sync_copy(d2h.at[pl.ds(wid * SROWP, SROWP)], d2)
    _zero_accum(e0, accum, sid, L)
    if not weighted:
        ones = jnp.ones((L,), f32)

        def orow(i, c):
            e0[i, :] = ones
            return c

        lax.fori_loop(0, MGSZ, orow, 0)

    def run_seg(s, nspg):
        sbase = ebase + s * (MGSZ * DSPG)

        def fire(k, b):
            if not weighted:
                return []
            return [pltpu.async_copy(w_rows.at[pl.ds(sbase + k * MGSZ, MGSZ)],
                                     bufs[b], gsems[b])]

        def store(k, b):
            r = s * DSPG + k
            return [pltpu.async_copy(bufs[b], accum.at[d2.at[r]],
                                     ssems[b], add=True)]

        _seg_pipeline(fire, lambda k, b: None, store, nspg)

    def seg(s, c):
        run_seg(s, DSPG)
        return c

    lax.fori_loop(0, DNSEG, seg, 0)
    run_seg(DNSEG, NSROW - DNSEG * DSPG)
    plsc.subcore_barrier()
    pltpu.sync_copy(accum.at[pl.ds(sid * NT, NT)],
                    out.at[pl.ds(cid * N_PAD + sid * NT, NT)])


def _make_sc_deg(weighted):
    scratch = [pltpu.VMEM((SROWP, SSZ), i32)]
    if weighted:
        scratch += ([pltpu.VMEM((MGSZ, L), f32)] * 3
                    + [pltpu.SemaphoreType.DMA] * 3)
    else:
        scratch += [pltpu.VMEM((MGSZ, L), f32)]
    scratch += ([pltpu.SemaphoreType.DMA] * 3
                + [pltpu.VMEM_SHARED((N_PAD, L), f32)])
    return functools.partial(
        pl.kernel,
        out_type=jax.ShapeDtypeStruct((NC * N_PAD, L), f32),
        mesh=_mesh(),
        scratch_types=scratch,
    )(functools.partial(_sc_deg_body, weighted))


_sc_deg_w = _make_sc_deg(True)
_sc_count = _make_sc_deg(False)


def _full(shape):
    return pl.BlockSpec(shape, lambda i: tuple(0 for _ in shape))


def _rows(b, width):
    return pl.BlockSpec((b, width), lambda i: (i, 0))


def _ln(z):
    mu = jnp.mean(z, axis=-1, keepdims=True)
    zc = z - mu
    var = jnp.mean(zc * zc, axis=-1, keepdims=True)
    return zc * lax.rsqrt(var + 1e-5)


def _silu(x):
    return x * jax.nn.sigmoid(x)


def _dot(a, b):
    return jnp.dot(a, b, preferred_element_type=f32)


def _tc_w2_body(wm_ref, dv_ref, out_ref):
    wm = wm_ref[...]
    dcl = jnp.clip(dv_ref[...], 0.0, 1.0)
    out_ref[...] = lax.dot_general(wm * dcl, wm, (((1,), (1,)), ((), ())),
                                   preferred_element_type=f32)


_tc_w2 = pl.pallas_call(
    _tc_w2_body,
    grid=(1,),
    in_specs=[_full((D, D)), _full((1, D))],
    out_specs=_full((D, D)),
    out_shape=jax.ShapeDtypeStruct((D, D), f32),
)


def _tc_a_body(z_ref, w1a_ref, w1b_ref, wg_ref, be1_ref, xa_ref, xb_ref, xw_ref):
    z = z_ref[...]
    xa_ref[...] = _dot(z, w1a_ref[...]) + be1_ref[...]
    xb_ref[...] = _dot(z, w1b_ref[...])
    xw_ref[...] = _dot(_ln(z), wg_ref[...])


_tc_a = pl.pallas_call(
    _tc_a_body,
    grid=(NB,),
    in_specs=[_rows(BN, D), _full((D, D)), _full((D, D)), _full((D, D)),
              _full((1, D))],
    out_specs=[_rows(BN, D)] * 3,
    out_shape=[jax.ShapeDtypeStruct((N, D), f32)] * 3,
)


def _tc_edge_body(hs_ref, we2_ref, be2_ref, w3_ref, be3_ref, ev_ref):
    h = _silu(hs_ref[...])
    u = _dot(h, we2_ref[...]) + be2_ref[...]
    u = _silu(u)
    t = jnp.sum(u * w3_ref[...], axis=-1, keepdims=True) + be3_ref[0]
    ev_ref[...] = jnp.broadcast_to(jax.nn.sigmoid(t), (t.shape[0], L))


_tc_edge = pl.pallas_call(
    _tc_edge_body,
    grid=(NEB,),
    in_specs=[_rows(EB, D), _full((D, D_IN)), _full((1, D_IN)),
              _full((1, D_IN)),
              pl.BlockSpec(memory_space=pltpu.SMEM)],
    out_specs=_rows(EB, L),
    out_shape=jax.ShapeDtypeStruct((E, L), f32),
)


def _deg_to_dinv(degp):
    deg = 1.0 + degp[0, :, 0:1] + degp[1, :, 0:1]
    return lax.rsqrt(deg)


def _tc_din_body(emb_ref, win_ref, degp_ref, xws_ref, dinv_ref):
    dinv = _deg_to_dinv(degp_ref[...])
    xws_ref[...] = dinv * _dot(emb_ref[...], win_ref[...])
    dinv_ref[...] = dinv


_tc_din = pl.pallas_call(
    _tc_din_body,
    grid=(NB,),
    in_specs=[_rows(BN, D_IN), _full((D_IN, D)),
              pl.BlockSpec((NC, BN, L), lambda i: (0, i, 0))],
    out_specs=[_rows(BN, D), _rows(BN, 1)],
    out_shape=[jax.ShapeDtypeStruct((N, D), f32),
               jax.ShapeDtypeStruct((N, 1), f32)],
)


def _tc_in_post_body(aggp_ref, xws_ref, dinv_ref, bin_ref, x0_ref):
    aggp = aggp_ref[...]
    t = dinv_ref[...] * (aggp[0] + aggp[1] + xws_ref[...]) + bin_ref[...]
    nrm = jnp.sqrt(jnp.sum(t * t, axis=-1, keepdims=True))
    x0_ref[...] = t / jnp.maximum(nrm, 1e-12)


_tc_in_post = pl.pallas_call(
    _tc_in_post_body,
    grid=(NB,),
    in_specs=[pl.BlockSpec((NC, BN, D), lambda i: (0, i, 0)),
              _rows(BN, D), _rows(BN, 1), _full((1, D))],
    out_specs=_rows(BN, D),
    out_shape=jax.ShapeDtypeStruct((N, D), f32),
)


def _tc_c_body(degp_ref, xw_ref, dinv_ref, xws_ref):
    dinv = _deg_to_dinv(degp_ref[...])
    dinv_ref[...] = dinv
    xws_ref[...] = dinv * xw_ref[...]


_tc_c = pl.pallas_call(
    _tc_c_body,
    grid=(NB,),
    in_specs=[pl.BlockSpec((NC, BN, L), lambda i: (0, i, 0)), _rows(BN, D)],
    out_specs=[_rows(BN, 1), _rows(BN, D)],
    out_shape=[jax.ShapeDtypeStruct((N, 1), f32),
               jax.ShapeDtypeStruct((N, D), f32)],
)


def _tc_d_body(stage, with_a, *refs):
    if stage == 0:
        (z_ref, cur_ref, aggp_ref, dinv_ref, xws_ref, al_ref, w2_ref, bg_ref,
         *rest) = refs
        acc_ref = None
    else:
        (z_ref, cur_ref, acc_ref, aggp_ref, dinv_ref, xws_ref, al_ref, w2_ref,
         bg_ref, *rest) = refs
    if with_a:
        w1a_ref, w1b_ref, wg_ref, be1_ref = rest[:4]
        outs = rest[4:]
        y_ref, accn_ref, xa_ref, xb_ref, xw_ref = outs
    else:
        y_ref, accn_ref = rest

    z = z_ref[...]
    aggp = aggp_ref[...]
    gn = dinv_ref[...] * (aggp[0] + aggp[1] + xws_ref[...]) + bg_ref[...] \
        - _ln(z)
    a2 = jax.nn.sigmoid(al_ref[...]) * 0.5
    k = a2 * gn - 2.0 * z + _dot(z, w2_ref[...])
    if stage == 0:
        accn = k
    elif stage in (1, 2):
        accn = acc_ref[...] + 2.0 * k
    else:
        accn = acc_ref[...] + k
    accn_ref[...] = accn
    cur = cur_ref[...]
    if stage < 3:
        y = cur + (DT / 2.0 if stage < 2 else DT) * k
    else:
        y = cur + (DT / 6.0) * accn
    y_ref[...] = y
    if with_a:
        xa_ref[...] = _dot(y, w1a_ref[...]) + be1_ref[...]
        xb_ref[...] = _dot(y, w1b_ref[...])
        xw_ref[...] = _dot(_ln(y), wg_ref[...])


def _make_tc_d(stage, with_a):
    in_specs = [_rows(BN, D), _rows(BN, D)]
    if stage != 0:
        in_specs.append(_rows(BN, D))
    in_specs += [pl.BlockSpec((NC, BN, D), lambda i: (0, i, 0)),
                 _rows(BN, 1), _rows(BN, D), _rows(BN, 1),
                 _full((D, D)), _full((1, D))]
    n_out = 2
    if with_a:
        in_specs += [_full((D, D)), _full((D, D)), _full((D, D)),
                     _full((1, D))]
        n_out = 5
    return pl.pallas_call(
        functools.partial(_tc_d_body, stage, with_a),
        grid=(NB,),
        in_specs=in_specs,
        out_specs=[_rows(BN, D)] * n_out,
        out_shape=[jax.ShapeDtypeStruct((N, D), f32)] * n_out,
    )


_tc_d = {(s, wa): _make_tc_d(s, wa)
         for s in range(4) for wa in (True, False)}


def _tc_out_pre_body(sol_ref, dinv_ref, ts_ref):
    ts_ref[...] = dinv_ref[...] * _silu(sol_ref[...])


_tc_out_pre = pl.pallas_call(
    _tc_out_pre_body,
    grid=(NB,),
    in_specs=[_rows(BN, D), _rows(BN, 1)],
    out_specs=_rows(BN, D),
    out_shape=jax.ShapeDtypeStruct((N, D), f32),
)


def _tc_out_post_body(aggp_ref, ts_ref, wout_ref, dinv_ref, bout_ref, y_ref):
    aggp = aggp_ref[...]
    t = aggp[0] + aggp[1] + ts_ref[...]
    y_ref[...] = dinv_ref[...] * _dot(t, wout_ref[...]) + bout_ref[...]


_tc_out_post = pl.pallas_call(
    _tc_out_post_body,
    grid=(NB,),
    in_specs=[pl.BlockSpec((NC, BN, D), lambda i: (0, i, 0)),
              _rows(BN, D), _full((D, D_IN)), _rows(BN, 1), _full((1, D_IN))],
    out_specs=_rows(BN, D_IN),
    out_shape=jax.ShapeDtypeStruct((N, D_IN), f32),
)


def kernel(node_embeddings, edge_index, W_in, b_in, W_out, b_out, W_g, b_g,
           W_e1, b_e1, W_e2, b_e2, W_e3, b_e3, alpha, w_mat, d_vec):
    src = edge_index[0]
    dst = edge_index[1]
    d2h = jnp.pad(dst.reshape(NW, NSROW, SSZ),
                  ((0, 0), (0, SROWP - NSROW), (0, 0))).reshape(NW * SROWP, SSZ)
    w1a = W_e1[:D]
    w1b = W_e1[D:]
    be1 = b_e1.reshape(1, D)
    be2 = b_e2.reshape(1, D_IN)
    w3 = W_e3.reshape(1, D_IN)
    bg = b_g.reshape(1, D)
    bi = b_in.reshape(1, D)
    bo = b_out.reshape(1, D_IN)
    al = alpha.reshape(N, 1)
    dv = d_vec.reshape(1, D)

    w2 = _tc_w2(w_mat, dv)
    deg0p = _sc_count(d2h).reshape(NC, N_PAD, L)
    xws0, dinv0 = _tc_din(node_embeddings, W_in, deg0p)
    agg0p = _sc_msg_plain(xws0, src, d2h).reshape(NC, N_PAD, D)
    x0 = _tc_in_post(agg0p, xws0, dinv0, bi)

    xa, xb, xw = _tc_a(x0, w1a, w1b, W_g, be1)
    sols = []
    cur = x0
    z = x0
    acc = None
    for step in range(2):
        for stage in range(4):
            hsum = _sc_gather_add(xa, xb, src, dst)
            ev_rows = _tc_edge(hsum, W_e2, be2, w3, b_e3)
            degp = _sc_deg_w(ev_rows, d2h).reshape(NC, N_PAD, L)
            dinv, xws = _tc_c(degp, xw)
            aggp = _sc_msg_ev(xws, ev_rows, src, d2h).reshape(NC, N_PAD, D)
            with_a = not (step == 1 and stage == 3)
            args = [z, cur] + ([] if stage == 0 else [acc]) + \
                [aggp, dinv, xws, al, w2, bg]
            if with_a:
                args += [w1a, w1b, W_g, be1]
                y, acc, xa, xb, xw = _tc_d[(stage, True)](*args)
            else:
                y, acc = _tc_d[(stage, False)](*args)
            z = y
            if stage == 3:
                cur = y
        sols.append(cur)

    outs = []
    for i in range(2):
        ts = _tc_out_pre(sols[i], dinv0)
        aggo = _sc_msg_plain(ts, src, d2h).reshape(NC, N_PAD, D)
        outs.append(_tc_out_post(aggo, ts, W_out, dinv0, bo))
    return (jnp.stack(outs, axis=0), outs[-1])
